# baseline (speedup 1.0000x reference)
import jax
import jax.numpy as jnp
from jax.experimental import pallas as pl


def _pass1_kernel(a_ref, x_ref, xb_ref, y_ref, qa_ref):
    a = a_ref[...]
    y_ref[...] = jnp.dot(a.astype(jnp.bfloat16), x_ref[...],
                         preferred_element_type=jnp.float32) + xb_ref[...]
    qa_ref[...] = (jnp.round(a * 255.0) - 128.0).astype(jnp.int8)


def kernel(x, A):
    n, d = x.shape
    bm = 400
    nm = n // bm
    x16 = x.astype(jnp.bfloat16)
    y = pl.pallas_call(
        _pass1_kernel,
        grid=(nm,),
        in_specs=[
            pl.BlockSpec((bm, n), lambda m: (m, 0)),
            pl.BlockSpec((n, d), lambda m: (0, 0)),
            pl.BlockSpec((bm, d), lambda m: (m, 0)),
        ],
        out_specs=[
            pl.BlockSpec((bm, d), lambda m: (m, 0)),
            pl.BlockSpec((bm, n), lambda m: (m, 0)),
        ],
        out_shape=[
            jax.ShapeDtypeStruct((n, d), jnp.float32),
            jax.ShapeDtypeStruct((n, n), jnp.int8),
        ],
    )(A, x16, x)
    return y
